# Initial kernel scaffold; baseline (speedup 1.0000x reference)
import jax
import jax.numpy as jnp
from jax.experimental import pallas as pl


def kernel(x, assign, W_proj, gate, hops):
    raise NotImplementedError("write your pallas kernel here")


# speedup vs baseline: 8.3229x; 8.3229x over previous
import jax
import jax.numpy as jnp
import numpy as np
from jax.experimental import pallas as pl
from jax.experimental.pallas import tpu as pltpu

B, T, D, A = 4, 8192, 768, 256
RINGS = 1
TB = 1024
NT = T // TB


def _neighbor_w():
    w = np.zeros((A, A), dtype=np.float32)
    for a in range(A):
        lo, hi = max(0, a - RINGS), min(A, a + RINGS + 1)
        w[a, lo:hi] = 1.0 / (hi - lo)
    return jnp.asarray(w)


def _pool_body(assign_ref, x_ref, sums_ref, counts_ref):
    b = pl.program_id(0)
    tb = pl.program_id(1)
    a_ids = assign_ref[0, 0, :]
    rows = jax.lax.broadcasted_iota(jnp.int32, (A, TB), 0)
    onehot_t = (rows == a_ids[None, :]).astype(jnp.bfloat16)
    x_blk = x_ref[0].astype(jnp.bfloat16)
    partial = jax.lax.dot_general(
        onehot_t, x_blk, (((1,), (0,)), ((), ())),
        preferred_element_type=jnp.float32)

    @pl.when(tb == 0)
    def _():
        sums_ref[0] = partial

    @pl.when(tb != 0)
    def _():
        sums_ref[0] += partial

    cpart = jnp.sum((rows == a_ids[None, :]).astype(jnp.float32), axis=1)

    @pl.when((b == 0) & (tb == 0))
    def _():
        counts_ref[0, :] = cpart

    @pl.when((b == 0) & (tb != 0))
    def _():
        counts_ref[0, :] += cpart


def _mid_body(gate_ref, hops_ref, sums_ref, counts_ref, wn_ref, wp_ref,
              scaled_ref):
    inv = 1.0 / jnp.maximum(counts_ref[0, :], 1.0)
    g = jnp.tanh(gate_ref[0])
    wn = wn_ref[...]
    wp = wp_ref[...]
    nhops = jnp.maximum(1, hops_ref[0])
    for b in range(B):
        anchor = sums_ref[b] * inv[:, None]
        agg = jax.lax.fori_loop(
            0, nhops,
            lambda _, a: jnp.dot(wn, a, preferred_element_type=jnp.float32),
            anchor)
        proj = jax.lax.dot_general(
            agg, wp, (((1,), (1,)), ((), ())),
            preferred_element_type=jnp.float32)
        scaled_ref[b] = (proj * g).astype(jnp.bfloat16)


def _bcast_body(assign_ref, x_ref, scaled_ref, out_ref):
    a_ids = assign_ref[0, 0, :]
    cols = jax.lax.broadcasted_iota(jnp.int32, (TB, A), 1)
    onehot = (cols == a_ids[:, None]).astype(jnp.bfloat16)
    fb = jax.lax.dot_general(
        onehot, scaled_ref[0], (((1,), (0,)), ((), ())),
        preferred_element_type=jnp.float32)
    out_ref[0] = x_ref[0] + fb


def kernel(x, assign, W_proj, gate, hops):
    assign3 = assign.astype(jnp.int32).reshape(NT, 1, TB)

    sums, counts = pl.pallas_call(
        _pool_body,
        grid=(B, NT),
        in_specs=[
            pl.BlockSpec((1, 1, TB), lambda b, t: (t, 0, 0)),
            pl.BlockSpec((1, TB, D), lambda b, t: (b, t, 0)),
        ],
        out_specs=[
            pl.BlockSpec((1, A, D), lambda b, t: (b, 0, 0)),
            pl.BlockSpec((1, A), lambda b, t: (0, 0)),
        ],
        out_shape=[
            jax.ShapeDtypeStruct((B, A, D), jnp.float32),
            jax.ShapeDtypeStruct((1, A), jnp.float32),
        ],
    )(assign3, x)

    wn = _neighbor_w()
    gate_s = jnp.reshape(jnp.asarray(gate, jnp.float32), (1,))
    hops_s = jnp.reshape(jnp.asarray(hops, jnp.int32), (1,))
    scaled = pl.pallas_call(
        _mid_body,
        in_specs=[
            pl.BlockSpec(memory_space=pltpu.SMEM),
            pl.BlockSpec(memory_space=pltpu.SMEM),
            pl.BlockSpec((B, A, D), lambda: (0, 0, 0)),
            pl.BlockSpec((1, A), lambda: (0, 0)),
            pl.BlockSpec((A, A), lambda: (0, 0)),
            pl.BlockSpec((D, D), lambda: (0, 0)),
        ],
        out_specs=pl.BlockSpec((B, A, D), lambda: (0, 0, 0)),
        out_shape=jax.ShapeDtypeStruct((B, A, D), jnp.bfloat16),
    )(gate_s, hops_s, sums, counts, wn, W_proj)

    out = pl.pallas_call(
        _bcast_body,
        grid=(B, NT),
        in_specs=[
            pl.BlockSpec((1, 1, TB), lambda b, t: (t, 0, 0)),
            pl.BlockSpec((1, TB, D), lambda b, t: (b, t, 0)),
            pl.BlockSpec((1, A, D), lambda b, t: (b, 0, 0)),
        ],
        out_specs=pl.BlockSpec((1, TB, D), lambda b, t: (b, t, 0)),
        out_shape=jax.ShapeDtypeStruct((B, T, D), jnp.float32),
    )(assign3, x, scaled)

    return out
